# baseline (speedup 1.0000x reference)
import functools

import jax
import jax.numpy as jnp
from jax.experimental import pallas as pl
from jax.experimental.pallas import tpu as pltpu

LANE = 128
SUBLANE = 8
TC = 16
NUM_LABELS = 17
OUT_DTYPE = jnp.bfloat16


def _round_up(x, m):
    return ((x + m - 1) // m) * m


def _vmem_limit_bytes():
    return 48 * 1024 * 1024


def _bilstm_kernel(x_ref, wih_ref, b_ref, whh_ref, len_ref, y_ref,
                   gx_sc, h_sc, c_sc, *, hidden, batch, tc, nc, mask_y):
    d = pl.program_id(0)
    s = pl.program_id(1)
    kin = x_ref.shape[0]

    @pl.when(s == 0)
    def _init():
        h_sc[...] = jnp.zeros_like(h_sc)
        c_sc[...] = jnp.zeros_like(c_sc)

    @pl.when(s < nc)
    def _proj():
        acc = jnp.dot(x_ref[0], wih_ref[0], preferred_element_type=jnp.float32)
        for k in range(1, kin):
            acc = acc + jnp.dot(x_ref[k], wih_ref[k],
                                preferred_element_type=jnp.float32)
        gx_sc[s % 2] = acc + b_ref[...]

    lens = len_ref[...]
    whh = whh_ref[...]

    @pl.when(s > 0)
    def _recurrence():
        buf = (s - 1) % 2
        base = ((1 - d) * (s - 1) + d * (nc - s)) * tc
        h, c = h_sc[...], c_sc[...]
        for k in range(tc):
            r = (1 - d) * k + d * (tc - 1 - k)
            r0 = r * batch
            t_glob = base + r
            gates = gx_sc[buf, pl.ds(r0, batch), :] + jnp.dot(
                h, whh, preferred_element_type=jnp.float32)
            i_g = jax.nn.sigmoid(gates[:, 0 * hidden:1 * hidden])
            f_g = jax.nn.sigmoid(gates[:, 1 * hidden:2 * hidden])
            g_g = jnp.tanh(gates[:, 2 * hidden:3 * hidden])
            o_g = jax.nn.sigmoid(gates[:, 3 * hidden:4 * hidden])
            c_new = f_g * c + i_g * g_g
            h_new = (o_g * jnp.tanh(c_new)).astype(jnp.bfloat16)
            valid = lens > t_glob
            if mask_y:
                y = jnp.where(valid, h_new, jnp.zeros_like(h_new))
            else:
                y = h_new
            y_ref[pl.ds(r0, batch), :] = y
            h = jnp.where(valid, h_new, h)
            c = jnp.where(valid, c_new, c)
        h_sc[...] = h
        c_sc[...] = c


def _bilstm_layer(x, wih, b, whh, lens2d, *, seq_len, batch, tc, mask_y):
    kin, m_rows, din = x.shape
    g = wih.shape[-1]
    hp = g // 4
    nc = seq_len // tc
    rows = tc * batch

    def xmap(d, s):
        cl = jnp.minimum(s, nc - 1)
        return (0, (1 - d) * cl + d * (nc - 1 - cl), 0)

    def ymap(d, s):
        cp = jnp.maximum(s - 1, 0)
        return (d, (1 - d) * cp + d * (nc - 1 - cp), 0)

    grid_spec = pltpu.PrefetchScalarGridSpec(
        num_scalar_prefetch=0,
        grid=(2, nc + 1),
        in_specs=[
            pl.BlockSpec((kin, rows, din), xmap),
            pl.BlockSpec((None, kin, din, g), lambda d, s: (d, 0, 0, 0)),
            pl.BlockSpec((None, 1, g), lambda d, s: (d, 0, 0)),
            pl.BlockSpec((None, hp, g), lambda d, s: (d, 0, 0)),
            pl.BlockSpec((batch, 1), lambda d, s: (0, 0)),
        ],
        out_specs=pl.BlockSpec((None, rows, hp), ymap),
        scratch_shapes=[
            pltpu.VMEM((2, rows, g), jnp.float32),
            pltpu.VMEM((batch, hp), jnp.bfloat16),
            pltpu.VMEM((batch, hp), jnp.float32),
        ],
    )
    return pl.pallas_call(
        functools.partial(_bilstm_kernel, hidden=hp, batch=batch, tc=tc,
                          nc=nc, mask_y=mask_y),
        out_shape=jax.ShapeDtypeStruct((2, m_rows, hp), OUT_DTYPE),
        grid_spec=grid_spec,
        compiler_params=pltpu.CompilerParams(
            dimension_semantics=("parallel", "arbitrary"),
            vmem_limit_bytes=_vmem_limit_bytes()),
    )(x, wih, b, whh, lens2d)


def _proj_kernel(x_ref, w_ref, b_ref, o_ref):
    acc = jnp.dot(x_ref[0], w_ref[0], preferred_element_type=jnp.float32)
    acc = acc + jnp.dot(x_ref[1], w_ref[1], preferred_element_type=jnp.float32)
    o_ref[...] = acc + b_ref[...]


def _output_projection(x, w, b, tm=512):
    _, m_rows, hp = x.shape
    lp = w.shape[-1]
    while m_rows % tm:
        tm //= 2
    grid_spec = pltpu.PrefetchScalarGridSpec(
        num_scalar_prefetch=0,
        grid=(m_rows // tm,),
        in_specs=[
            pl.BlockSpec((2, tm, hp), lambda m: (0, m, 0)),
            pl.BlockSpec((2, hp, lp), lambda m: (0, 0, 0)),
            pl.BlockSpec((1, lp), lambda m: (0, 0)),
        ],
        out_specs=pl.BlockSpec((tm, lp), lambda m: (m, 0)),
    )
    return pl.pallas_call(
        _proj_kernel,
        out_shape=jax.ShapeDtypeStruct((m_rows, lp), jnp.float32),
        grid_spec=grid_spec,
        compiler_params=pltpu.CompilerParams(
            dimension_semantics=("parallel",),
            vmem_limit_bytes=_vmem_limit_bytes()),
    )(x, w, b)


def kernel(token_ids, lengths, embedding, lstm0_w_ih, lstm0_w_hh, lstm0_b,
           lstm1_w_ih, lstm1_w_hh, lstm1_b, w_out, b_out):
    B, T = token_ids.shape
    tc = TC
    Bp = _round_up(max(B, SUBLANE), SUBLANE)
    Tp = _round_up(T, tc)

    ids = token_ids
    lens = lengths.astype(jnp.int32)
    if (Bp, Tp) != (B, T):
        ids = jnp.zeros((Bp, Tp), token_ids.dtype).at[:B, :T].set(token_ids)
        lens = jnp.zeros((Bp,), jnp.int32).at[:B].set(lens)
    lens2d = lens.reshape(Bp, 1)

    emb = jnp.take(embedding, ids.T, axis=0)
    x = emb.reshape(1, Tp * Bp, emb.shape[-1])

    x = _bilstm_layer(x, lstm0_w_ih, lstm0_b, lstm0_w_hh, lens2d,
                      seq_len=Tp, batch=Bp, tc=tc, mask_y=False)
    x = _bilstm_layer(x, lstm1_w_ih, lstm1_b, lstm1_w_hh, lens2d,
                      seq_len=Tp, batch=Bp, tc=tc, mask_y=True)

    logits_p = _output_projection(x, w_out, b_out)
    logits = logits_p.reshape(Tp, Bp, -1)[:T, :B, :NUM_LABELS]
    return jnp.transpose(logits, (1, 0, 2))


# speedup vs baseline: 1.3798x; 1.3798x over previous
import functools

import jax
import jax.numpy as jnp
from jax.experimental import pallas as pl
from jax.experimental.pallas import tpu as pltpu

LANE = 128
SUBLANE = 8
TC = 16
NUM_LABELS = 17
OUT_DTYPE = jnp.bfloat16


def _round_up(x, m):
    return ((x + m - 1) // m) * m


def _vmem_limit_bytes():
    return 48 * 1024 * 1024


def _layer_kernel(xf_ref, xb_ref, wih_ref, b_ref, whh_ref, len_ref,
                  yf_ref, yb_ref, gxf_sc, gxb_sc, hf_sc, cf_sc, hb_sc, cb_sc,
                  *, hidden, batch, tc, nc, mask_y):
    s = pl.program_id(0)
    kin = xf_ref.shape[0]

    @pl.when(s == 0)
    def _init():
        hf_sc[...] = jnp.zeros_like(hf_sc)
        cf_sc[...] = jnp.zeros_like(cf_sc)
        hb_sc[...] = jnp.zeros_like(hb_sc)
        cb_sc[...] = jnp.zeros_like(cb_sc)

    @pl.when(s < nc)
    def _proj():
        af = jnp.dot(xf_ref[0], wih_ref[0, 0],
                     preferred_element_type=jnp.float32)
        ab = jnp.dot(xb_ref[0], wih_ref[1, 0],
                     preferred_element_type=jnp.float32)
        for k in range(1, kin):
            af = af + jnp.dot(xf_ref[k], wih_ref[0, k],
                              preferred_element_type=jnp.float32)
            ab = ab + jnp.dot(xb_ref[k], wih_ref[1, k],
                              preferred_element_type=jnp.float32)
        gxf_sc[s % 2] = af + b_ref[0]
        gxb_sc[s % 2] = ab + b_ref[1]

    lens = len_ref[...]
    whh_f = whh_ref[0]
    whh_b = whh_ref[1]

    @pl.when(s > 0)
    def _recurrence():
        buf = (s - 1) % 2
        base_f = (s - 1) * tc
        base_b = (nc - s) * tc
        hf, cf = hf_sc[...], cf_sc[...]
        hb, cb = hb_sc[...], cb_sc[...]

        def cell(gx_sc, whh, r, t_glob, h, c, mask_carry, y_ref):
            r0 = r * batch
            gates = gx_sc[buf, r0:r0 + batch, :] + jnp.dot(
                h, whh, preferred_element_type=jnp.float32)
            i_g = jax.nn.sigmoid(gates[:, 0 * hidden:1 * hidden])
            f_g = jax.nn.sigmoid(gates[:, 1 * hidden:2 * hidden])
            g_g = jnp.tanh(gates[:, 2 * hidden:3 * hidden])
            o_g = jax.nn.sigmoid(gates[:, 3 * hidden:4 * hidden])
            c_new = f_g * c + i_g * g_g
            h_new = (o_g * jnp.tanh(c_new)).astype(jnp.bfloat16)
            if mask_y or mask_carry:
                valid = lens > t_glob
            if mask_y:
                y = jnp.where(valid, h_new, jnp.zeros_like(h_new))
            else:
                y = h_new
            y_ref[r0:r0 + batch, :] = y
            if mask_carry:
                h_new2 = jnp.where(valid, h_new, h)
                c_new2 = jnp.where(valid, c_new, c)
                return h_new2, c_new2
            return h_new, c_new

        for k in range(tc):
            hf, cf = cell(gxf_sc, whh_f, k, base_f + k, hf, cf,
                          False, yf_ref)
            hb, cb = cell(gxb_sc, whh_b, tc - 1 - k, base_b + tc - 1 - k,
                          hb, cb, True, yb_ref)
        hf_sc[...] = hf
        cf_sc[...] = cf
        hb_sc[...] = hb
        cb_sc[...] = cb


def _bilstm_layer_il(xf, xb, wih, b, whh, lens2d, *, seq_len, batch, tc,
                     mask_y):
    kin, m_rows, din = xf.shape
    g = wih.shape[-1]
    hp = g // 4
    nc = seq_len // tc
    rows = tc * batch

    def fmap(s):
        return (0, jnp.minimum(s, nc - 1), 0)

    def bmap(s):
        return (0, nc - 1 - jnp.minimum(s, nc - 1), 0)

    def yfmap(s):
        return (jnp.maximum(s - 1, 0), 0)

    def ybmap(s):
        return (nc - 1 - jnp.maximum(s - 1, 0), 0)

    grid_spec = pltpu.PrefetchScalarGridSpec(
        num_scalar_prefetch=0,
        grid=(nc + 1,),
        in_specs=[
            pl.BlockSpec((kin, rows, din), fmap),
            pl.BlockSpec((kin, rows, din), bmap),
            pl.BlockSpec((2, kin, din, g), lambda s: (0, 0, 0, 0)),
            pl.BlockSpec((2, 1, g), lambda s: (0, 0, 0)),
            pl.BlockSpec((2, hp, g), lambda s: (0, 0, 0)),
            pl.BlockSpec((batch, 1), lambda s: (0, 0)),
        ],
        out_specs=[
            pl.BlockSpec((rows, hp), yfmap),
            pl.BlockSpec((rows, hp), ybmap),
        ],
        scratch_shapes=[
            pltpu.VMEM((2, rows, g), jnp.float32),
            pltpu.VMEM((2, rows, g), jnp.float32),
            pltpu.VMEM((batch, hp), jnp.bfloat16),
            pltpu.VMEM((batch, hp), jnp.float32),
            pltpu.VMEM((batch, hp), jnp.bfloat16),
            pltpu.VMEM((batch, hp), jnp.float32),
        ],
    )
    return pl.pallas_call(
        functools.partial(_layer_kernel, hidden=hp, batch=batch, tc=tc,
                          nc=nc, mask_y=mask_y),
        out_shape=[jax.ShapeDtypeStruct((m_rows, hp), OUT_DTYPE),
                   jax.ShapeDtypeStruct((m_rows, hp), OUT_DTYPE)],
        grid_spec=grid_spec,
        compiler_params=pltpu.CompilerParams(
            dimension_semantics=("arbitrary",),
            vmem_limit_bytes=_vmem_limit_bytes()),
    )(xf, xb, wih, b, whh, lens2d)


def _proj_kernel(xf_ref, xb_ref, w_ref, b_ref, o_ref):
    acc = jnp.dot(xf_ref[...], w_ref[0], preferred_element_type=jnp.float32)
    acc = acc + jnp.dot(xb_ref[...], w_ref[1],
                        preferred_element_type=jnp.float32)
    o_ref[...] = acc + b_ref[...]


def _output_projection(xf, xb, w, b, tm=1024):
    m_rows, hp = xf.shape
    lp = w.shape[-1]
    while m_rows % tm:
        tm //= 2
    grid_spec = pltpu.PrefetchScalarGridSpec(
        num_scalar_prefetch=0,
        grid=(m_rows // tm,),
        in_specs=[
            pl.BlockSpec((tm, hp), lambda m: (m, 0)),
            pl.BlockSpec((tm, hp), lambda m: (m, 0)),
            pl.BlockSpec((2, hp, lp), lambda m: (0, 0, 0)),
            pl.BlockSpec((1, lp), lambda m: (0, 0)),
        ],
        out_specs=pl.BlockSpec((tm, lp), lambda m: (m, 0)),
    )
    return pl.pallas_call(
        _proj_kernel,
        out_shape=jax.ShapeDtypeStruct((m_rows, lp), jnp.float32),
        grid_spec=grid_spec,
        compiler_params=pltpu.CompilerParams(
            dimension_semantics=("arbitrary",),
            vmem_limit_bytes=_vmem_limit_bytes()),
    )(xf, xb, w, b)


def kernel(token_ids, lengths, embedding, lstm0_w_ih, lstm0_w_hh, lstm0_b,
           lstm1_w_ih, lstm1_w_hh, lstm1_b, w_out, b_out):
    B, T = token_ids.shape
    tc = TC
    Bp = _round_up(max(B, SUBLANE), SUBLANE)
    Tp = _round_up(T, tc)
    M = Tp * Bp

    ids = token_ids
    lens = lengths.astype(jnp.int32)
    if (Bp, Tp) != (B, T):
        ids = jnp.zeros((Bp, Tp), token_ids.dtype).at[:B, :T].set(token_ids)
        lens = jnp.zeros((Bp,), jnp.int32).at[:B].set(lens)
    lens2d = lens.reshape(Bp, 1)

    emb = jnp.take(embedding, ids.T, axis=0)
    x = emb.reshape(1, M, emb.shape[-1])

    yf, yb = _bilstm_layer_il(x, x, lstm0_w_ih, lstm0_b, lstm0_w_hh, lens2d,
                              seq_len=Tp, batch=Bp, tc=tc, mask_y=False)
    x1 = jnp.stack([yf, yb])
    yf, yb = _bilstm_layer_il(x1, x1, lstm1_w_ih, lstm1_b, lstm1_w_hh, lens2d,
                              seq_len=Tp, batch=Bp, tc=tc, mask_y=True)

    logits_p = _output_projection(yf, yb, w_out, b_out)
    logits = logits_p.reshape(Tp, Bp, -1)[:T, :B, :NUM_LABELS]
    return jnp.transpose(logits, (1, 0, 2))


# speedup vs baseline: 1.4102x; 1.0220x over previous
import functools

import jax
import jax.numpy as jnp
from jax.experimental import pallas as pl
from jax.experimental.pallas import tpu as pltpu

LANE = 128
SUBLANE = 8
TC = 32
NUM_LABELS = 17
OUT_DTYPE = jnp.bfloat16


def _round_up(x, m):
    return ((x + m - 1) // m) * m


def _vmem_limit_bytes():
    return 48 * 1024 * 1024


def _layer_kernel(xf_ref, xb_ref, wih_ref, b_ref, whh_ref, len_ref,
                  yf_ref, yb_ref, gxf_sc, gxb_sc, hf_sc, cf_sc, hb_sc, cb_sc,
                  *, hidden, batch, tc, nc, mask_y):
    s = pl.program_id(0)
    kin = xf_ref.shape[0]

    @pl.when(s == 0)
    def _init():
        hf_sc[...] = jnp.zeros_like(hf_sc)
        cf_sc[...] = jnp.zeros_like(cf_sc)
        hb_sc[...] = jnp.zeros_like(hb_sc)
        cb_sc[...] = jnp.zeros_like(cb_sc)

    @pl.when(s < nc)
    def _proj():
        af = jnp.dot(xf_ref[0], wih_ref[0, 0],
                     preferred_element_type=jnp.float32)
        ab = jnp.dot(xb_ref[0], wih_ref[1, 0],
                     preferred_element_type=jnp.float32)
        for k in range(1, kin):
            af = af + jnp.dot(xf_ref[k], wih_ref[0, k],
                              preferred_element_type=jnp.float32)
            ab = ab + jnp.dot(xb_ref[k], wih_ref[1, k],
                              preferred_element_type=jnp.float32)
        gxf_sc[s % 2] = af + b_ref[0]
        gxb_sc[s % 2] = ab + b_ref[1]

    lens = len_ref[...]
    whh_f = whh_ref[0]
    whh_b = whh_ref[1]

    @pl.when(s > 0)
    def _recurrence():
        buf = (s - 1) % 2
        base_f = (s - 1) * tc
        base_b = (nc - s) * tc
        hf, cf = hf_sc[...], cf_sc[...]
        hb, cb = hb_sc[...], cb_sc[...]

        def cell(gx_sc, whh, r, t_glob, h, c, mask_carry, y_ref):
            r0 = r * batch
            gates = gx_sc[buf, r0:r0 + batch, :] + jnp.dot(
                h, whh, preferred_element_type=jnp.float32)
            i_g = jax.nn.sigmoid(gates[:, 0 * hidden:1 * hidden])
            f_g = jax.nn.sigmoid(gates[:, 1 * hidden:2 * hidden])
            g_g = jnp.tanh(gates[:, 2 * hidden:3 * hidden])
            o_g = jax.nn.sigmoid(gates[:, 3 * hidden:4 * hidden])
            c_new = f_g * c + i_g * g_g
            h_new = (o_g * jnp.tanh(c_new)).astype(jnp.bfloat16)
            if mask_y or mask_carry:
                valid = lens > t_glob
            if mask_y:
                y = jnp.where(valid, h_new, jnp.zeros_like(h_new))
            else:
                y = h_new
            y_ref[r0:r0 + batch, :] = y
            if mask_carry:
                h_new2 = jnp.where(valid, h_new, h)
                c_new2 = jnp.where(valid, c_new, c)
                return h_new2, c_new2
            return h_new, c_new

        for k in range(tc):
            hf, cf = cell(gxf_sc, whh_f, k, base_f + k, hf, cf,
                          False, yf_ref)
            hb, cb = cell(gxb_sc, whh_b, tc - 1 - k, base_b + tc - 1 - k,
                          hb, cb, True, yb_ref)
        hf_sc[...] = hf
        cf_sc[...] = cf
        hb_sc[...] = hb
        cb_sc[...] = cb


def _bilstm_layer_il(xf, xb, wih, b, whh, lens2d, *, seq_len, batch, tc,
                     mask_y):
    kin, m_rows, din = xf.shape
    g = wih.shape[-1]
    hp = g // 4
    nc = seq_len // tc
    rows = tc * batch

    def fmap(s):
        return (0, jnp.minimum(s, nc - 1), 0)

    def bmap(s):
        return (0, nc - 1 - jnp.minimum(s, nc - 1), 0)

    def yfmap(s):
        return (jnp.maximum(s - 1, 0), 0)

    def ybmap(s):
        return (nc - 1 - jnp.maximum(s - 1, 0), 0)

    grid_spec = pltpu.PrefetchScalarGridSpec(
        num_scalar_prefetch=0,
        grid=(nc + 1,),
        in_specs=[
            pl.BlockSpec((kin, rows, din), fmap),
            pl.BlockSpec((kin, rows, din), bmap),
            pl.BlockSpec((2, kin, din, g), lambda s: (0, 0, 0, 0)),
            pl.BlockSpec((2, 1, g), lambda s: (0, 0, 0)),
            pl.BlockSpec((2, hp, g), lambda s: (0, 0, 0)),
            pl.BlockSpec((batch, 1), lambda s: (0, 0)),
        ],
        out_specs=[
            pl.BlockSpec((rows, hp), yfmap),
            pl.BlockSpec((rows, hp), ybmap),
        ],
        scratch_shapes=[
            pltpu.VMEM((2, rows, g), jnp.float32),
            pltpu.VMEM((2, rows, g), jnp.float32),
            pltpu.VMEM((batch, hp), jnp.bfloat16),
            pltpu.VMEM((batch, hp), jnp.float32),
            pltpu.VMEM((batch, hp), jnp.bfloat16),
            pltpu.VMEM((batch, hp), jnp.float32),
        ],
    )
    return pl.pallas_call(
        functools.partial(_layer_kernel, hidden=hp, batch=batch, tc=tc,
                          nc=nc, mask_y=mask_y),
        out_shape=[jax.ShapeDtypeStruct((m_rows, hp), OUT_DTYPE),
                   jax.ShapeDtypeStruct((m_rows, hp), OUT_DTYPE)],
        grid_spec=grid_spec,
        compiler_params=pltpu.CompilerParams(
            dimension_semantics=("arbitrary",),
            vmem_limit_bytes=_vmem_limit_bytes()),
    )(xf, xb, wih, b, whh, lens2d)


def _proj_kernel(xf_ref, xb_ref, w_ref, b_ref, o_ref):
    acc = jnp.dot(xf_ref[...], w_ref[0], preferred_element_type=jnp.float32)
    acc = acc + jnp.dot(xb_ref[...], w_ref[1],
                        preferred_element_type=jnp.float32)
    o_ref[...] = acc + b_ref[...]


def _output_projection(xf, xb, w, b, tm=1024):
    m_rows, hp = xf.shape
    lp = w.shape[-1]
    while m_rows % tm:
        tm //= 2
    grid_spec = pltpu.PrefetchScalarGridSpec(
        num_scalar_prefetch=0,
        grid=(m_rows // tm,),
        in_specs=[
            pl.BlockSpec((tm, hp), lambda m: (m, 0)),
            pl.BlockSpec((tm, hp), lambda m: (m, 0)),
            pl.BlockSpec((2, hp, lp), lambda m: (0, 0, 0)),
            pl.BlockSpec((1, lp), lambda m: (0, 0)),
        ],
        out_specs=pl.BlockSpec((tm, lp), lambda m: (m, 0)),
    )
    return pl.pallas_call(
        _proj_kernel,
        out_shape=jax.ShapeDtypeStruct((m_rows, lp), jnp.float32),
        grid_spec=grid_spec,
        compiler_params=pltpu.CompilerParams(
            dimension_semantics=("arbitrary",),
            vmem_limit_bytes=_vmem_limit_bytes()),
    )(xf, xb, w, b)


def kernel(token_ids, lengths, embedding, lstm0_w_ih, lstm0_w_hh, lstm0_b,
           lstm1_w_ih, lstm1_w_hh, lstm1_b, w_out, b_out):
    B, T = token_ids.shape
    tc = TC
    Bp = _round_up(max(B, SUBLANE), SUBLANE)
    Tp = _round_up(T, tc)
    M = Tp * Bp

    ids = token_ids
    lens = lengths.astype(jnp.int32)
    if (Bp, Tp) != (B, T):
        ids = jnp.zeros((Bp, Tp), token_ids.dtype).at[:B, :T].set(token_ids)
        lens = jnp.zeros((Bp,), jnp.int32).at[:B].set(lens)
    lens2d = lens.reshape(Bp, 1)

    emb = jnp.take(embedding, ids.T, axis=0)
    x = emb.reshape(1, M, emb.shape[-1])

    yf, yb = _bilstm_layer_il(x, x, lstm0_w_ih, lstm0_b, lstm0_w_hh, lens2d,
                              seq_len=Tp, batch=Bp, tc=tc, mask_y=False)
    x1 = jnp.stack([yf, yb])
    yf, yb = _bilstm_layer_il(x1, x1, lstm1_w_ih, lstm1_b, lstm1_w_hh, lens2d,
                              seq_len=Tp, batch=Bp, tc=tc, mask_y=True)

    logits_p = _output_projection(yf, yb, w_out, b_out)
    logits = logits_p.reshape(Tp, Bp, -1)[:T, :B, :NUM_LABELS]
    return jnp.transpose(logits, (1, 0, 2))
